# baseline (speedup 1.0000x reference)
import functools

import jax
import jax.numpy as jnp
from jax.experimental import pallas as pl
from jax.experimental.pallas import tpu as pltpu

_F32 = jnp.float32


_BF16 = jnp.bfloat16


def _bdot(a, b, dims):
    return jax.lax.dot_general(a.astype(_BF16), b.astype(_BF16), dims,
                               preferred_element_type=_F32)


_MM = (((1,), (0,)), ((), ()))


def _fdot(a, b, dims):
    return jax.lax.dot_general(a, b, dims, preferred_element_type=_F32)


def _xdot(a, b, dims):
    return jax.lax.dot_general(a, b, dims, precision=jax.lax.Precision.HIGHEST,
                               preferred_element_type=_F32)


def _graph(X):
    N, F = X.shape
    Xsq = X * X
    sq_col = jnp.sum(Xsq, axis=1, keepdims=True)
    ones_row = jnp.ones((1, F), _F32)
    sq_row = _xdot(ones_row, Xsq, (((1,), (1,)), ((), ())))
    X_bf = X.astype(_BF16)
    inner = -2.0 * _fdot(X_bf, X_bf, (((1,), (1,)), ((), ())))
    adj = sq_col + inner + sq_row
    Wg = jnp.exp(-adj)

    rows = jax.lax.broadcasted_iota(jnp.int32, (N, N), 0)
    cols = jax.lax.broadcasted_iota(jnp.int32, (N, N), 1)
    A = jnp.where(rows == cols, 0.0, Wg)

    d_col = jnp.sum(A, axis=1, keepdims=True)
    dinv_col = jnp.where(d_col > 0, 1.0 / jnp.sqrt(jnp.where(d_col > 0, d_col, 1.0)), 0.0)
    dinv_row = jnp.transpose(dinv_col)
    return (A * dinv_col * dinv_row).astype(_BF16)


def _cheb(X, An_bf, wk_ref, bias_ref, K):
    X_bf = X.astype(_BF16)
    wk_bf = [wk_ref[k].astype(_BF16) for k in range(K)]
    out = _fdot(X_bf, wk_bf[0], _MM)
    if K > 1:
        Tx1 = -_fdot(An_bf, X_bf, _MM)
        Tx1_bf = Tx1.astype(_BF16)
        out = out + _fdot(Tx1_bf, wk_bf[1], _MM)
        Tx0 = X
        for k in range(2, K):
            Tx2 = -2.0 * _fdot(An_bf, Tx1_bf, _MM) - Tx0
            Tx2_bf = Tx2.astype(_BF16)
            out = out + _fdot(Tx2_bf, wk_bf[k], _MM)
            Tx0, Tx1_bf = Tx1, Tx2_bf
            Tx1 = Tx2
    return jnp.maximum(out + bias_ref[...], 0.0)


def _mreg_update(out, An_bf, mreg, reg_ref, b, nb):
    out_bf = out.astype(_BF16)
    Lout = out_bf.astype(_F32) - _fdot(An_bf, out_bf, _MM)
    contrib = _fdot(out_bf, Lout.astype(_BF16), (((0,), (0,)), ((), ())))

    @pl.when(b == 0)
    def _():
        mreg[...] = contrib

    @pl.when(b > 0)
    def _():
        mreg[...] = mreg[...] + contrib

    @pl.when(b == nb - 1)
    def _():
        m = mreg[...]
        reg_ref[...] = jnp.broadcast_to(jnp.sqrt(jnp.sum(m * m)), (1, 1))


def _pipe_body(xg_ref, xc_ref, wk_ref, bias_ref, out_ref, reg_ref,
               an_db, mreg, *, K, nb, pool):
    i = pl.program_id(0)

    an_db[jax.lax.rem(i, 2)] = _graph(xg_ref[0])

    An_bf = an_db[jax.lax.rem(i + 1, 2)]
    out = _cheb(xc_ref[0], An_bf, wk_ref, bias_ref, K)
    if pool:
        out_ref[0] = jnp.max(out, axis=0, keepdims=True)
    else:
        out_ref[0] = out
    _mreg_update(out, An_bf, mreg, reg_ref, i - 1, nb)


def _head_body(p_ref, w1_ref, b1_ref, w2_ref, b2_ref, w3_ref, b3_ref,
               logits_ref, tail_ref):
    mm = lambda a, w: _bdot(a, w, (((1,), (0,)), ((), ())))
    h = jnp.maximum(mm(p_ref[...], w1_ref[...]) + b1_ref[...], 0.0)
    h = jnp.maximum(mm(h, w2_ref[...]) + b2_ref[...], 0.0)
    logits_ref[...] = mm(h, w3_ref[...]) + b3_ref[...]
    w1 = w1_ref[...]
    nw = jnp.sqrt(jnp.sum(w1 * w1))
    b1 = b1_ref[...]
    nb = jnp.sqrt(jnp.sum(b1 * b1))
    lane = jax.lax.broadcasted_iota(jnp.int32, (1, 8), 1)
    tail_ref[...] = jnp.where(lane % 2 == 0,
                              jnp.broadcast_to(nw, (1, 8)),
                              jnp.broadcast_to(nb, (1, 8)))


def _run_layer(x, wk, bias, last):
    B, N, Fin = x.shape
    K, _, Fout = wk.shape
    bias2 = bias.reshape(1, Fout)
    out_specs = [
        pl.BlockSpec((1, 1, Fout) if last else (1, N, Fout),
                     lambda i: (jnp.maximum(i - 1, 0), 0, 0)),
        pl.BlockSpec((1, 1), lambda i: (0, 0)),
    ]
    out_shape = [
        jax.ShapeDtypeStruct((B, 1, Fout) if last else (B, N, Fout), _F32),
        jax.ShapeDtypeStruct((1, 1), _F32),
    ]
    return pl.pallas_call(
        functools.partial(_pipe_body, K=K, nb=B, pool=last),
        grid=(B + 1,),
        in_specs=[
            pl.BlockSpec((1, N, Fin), lambda i: (jnp.minimum(i, B - 1), 0, 0)),
            pl.BlockSpec((1, N, Fin), lambda i: (jnp.maximum(i - 1, 0), 0, 0)),
            pl.BlockSpec((K, Fin, Fout), lambda i: (0, 0, 0)),
            pl.BlockSpec((1, Fout), lambda i: (0, 0)),
        ],
        out_specs=out_specs,
        out_shape=out_shape,
        scratch_shapes=[pltpu.VMEM((2, N, N), _BF16),
                        pltpu.VMEM((Fout, Fout), _F32)],
        compiler_params=pltpu.CompilerParams(
            dimension_semantics=("arbitrary",)),
    )(x, x, wk, bias2)


def kernel(x, conv1_w, conv1_b, conv2_w, conv2_b, conv3_w, conv3_b,
           fc1_w, fc1_b, fc2_w, fc2_b, fc3_w, fc3_b,
           batch, batch_size, nr_points):
    del batch, batch_size, nr_points
    out1, r1 = _run_layer(x, conv1_w, conv1_b, last=False)
    out2, r2 = _run_layer(out1, conv2_w, conv2_b, last=False)
    pooled, r3 = _run_layer(out2, conv3_w, conv3_b, last=True)
    pooled = pooled.reshape(pooled.shape[0], pooled.shape[2])

    Bn = pooled.shape[0]
    logits, tail = pl.pallas_call(
        _head_body,
        out_shape=[
            jax.ShapeDtypeStruct((Bn, fc3_w.shape[1]), _F32),
            jax.ShapeDtypeStruct((1, 8), _F32),
        ],
    )(pooled, fc1_w, fc1_b.reshape(1, -1), fc2_w, fc2_b.reshape(1, -1),
      fc3_w, fc3_b.reshape(1, -1))

    regs = jnp.concatenate([
        r1.reshape(1), r2.reshape(1), r3.reshape(1), tail[0, :6]])
    return logits, regs


# speedup vs baseline: 1.1865x; 1.1865x over previous
import functools

import jax
import jax.numpy as jnp
from jax.experimental import pallas as pl
from jax.experimental.pallas import tpu as pltpu

_F32 = jnp.float32


_BF16 = jnp.bfloat16


def _bdot(a, b, dims):
    return jax.lax.dot_general(a.astype(_BF16), b.astype(_BF16), dims,
                               preferred_element_type=_F32)


_MM = (((1,), (0,)), ((), ()))


def _fdot(a, b, dims):
    return jax.lax.dot_general(a, b, dims, preferred_element_type=_F32)


def _xdot(a, b, dims):
    return jax.lax.dot_general(a, b, dims, precision=jax.lax.Precision.HIGHEST,
                               preferred_element_type=_F32)


def _graph(X):
    N, F = X.shape
    Xsq = X * X
    sq_col = jnp.sum(Xsq, axis=1, keepdims=True)
    ones_row = jnp.ones((1, F), _F32)
    sq_row = _xdot(ones_row, Xsq, (((1,), (1,)), ((), ())))
    X_bf = X.astype(_BF16)
    inner = -2.0 * _fdot(X_bf, X_bf, (((1,), (1,)), ((), ())))
    adj = sq_col + inner + sq_row
    Wg = jnp.exp(-adj)

    rows = jax.lax.broadcasted_iota(jnp.int32, (N, N), 0)
    cols = jax.lax.broadcasted_iota(jnp.int32, (N, N), 1)
    A = jnp.where(rows == cols, 0.0, Wg)

    d_col = jnp.sum(A, axis=1, keepdims=True)
    dinv_col = jnp.where(d_col > 0, 1.0 / jnp.sqrt(jnp.where(d_col > 0, d_col, 1.0)), 0.0)
    dinv_row = jnp.transpose(dinv_col)
    return (A * dinv_col * dinv_row).astype(_BF16)


def _cheb(X, An_bf, wk_ref, bias_ref, K):
    X_bf = X.astype(_BF16)
    wk_bf = [wk_ref[k].astype(_BF16) for k in range(K)]
    out = _fdot(X_bf, wk_bf[0], _MM)
    if K > 1:
        Tx1 = -_fdot(An_bf, X_bf, _MM)
        Tx1_bf = Tx1.astype(_BF16)
        out = out + _fdot(Tx1_bf, wk_bf[1], _MM)
        Tx0 = X
        for k in range(2, K):
            Tx2 = -2.0 * _fdot(An_bf, Tx1_bf, _MM) - Tx0
            Tx2_bf = Tx2.astype(_BF16)
            out = out + _fdot(Tx2_bf, wk_bf[k], _MM)
            Tx0, Tx1_bf = Tx1, Tx2_bf
            Tx1 = Tx2
    return jnp.maximum(out + bias_ref[...], 0.0)


def _mreg_update(out, An_bf, mreg, reg_ref, b, nb):
    out_bf = out.astype(_BF16)
    Lout = out_bf.astype(_F32) - _fdot(An_bf, out_bf, _MM)
    contrib = _fdot(out_bf, Lout.astype(_BF16), (((0,), (0,)), ((), ())))

    @pl.when(b == 0)
    def _():
        mreg[...] = contrib

    @pl.when(b > 0)
    def _():
        mreg[...] = mreg[...] + contrib

    @pl.when(b == nb - 1)
    def _():
        m = mreg[...]
        reg_ref[...] = jnp.broadcast_to(jnp.sqrt(jnp.sum(m * m)), (1, 1))


def _layer_body(x_ref, wk_ref, bias_ref, out_ref, reg_ref, mreg, *, K, nb):
    b = pl.program_id(0)
    X = x_ref[0]
    An_bf = _graph(X)
    out = _cheb(X, An_bf, wk_ref, bias_ref, K)
    out_ref[0] = out
    _mreg_update(out, An_bf, mreg, reg_ref, b, nb)


def _layer3_body(x_ref, wk_ref, bias_ref,
                 w1_ref, b1_ref, w2_ref, b2_ref, w3_ref, b3_ref,
                 logits_ref, reg_ref, tail_ref, pooled_scr, mreg, *, K, nb):
    b = pl.program_id(0)
    X = x_ref[0]
    An_bf = _graph(X)
    out = _cheb(X, An_bf, wk_ref, bias_ref, K)
    pooled_scr[pl.ds(b, 1), :] = jnp.max(out, axis=0, keepdims=True)
    _mreg_update(out, An_bf, mreg, reg_ref, b, nb)

    @pl.when(b == nb - 1)
    def _():
        mm = lambda a, w: _bdot(a, w, _MM)
        h = jnp.maximum(mm(pooled_scr[...], w1_ref[...]) + b1_ref[...], 0.0)
        h = jnp.maximum(mm(h, w2_ref[...]) + b2_ref[...], 0.0)
        logits_ref[...] = mm(h, w3_ref[...]) + b3_ref[...]
        w1 = w1_ref[...]
        nw = jnp.sqrt(jnp.sum(w1 * w1))
        b1 = b1_ref[...]
        nb1 = jnp.sqrt(jnp.sum(b1 * b1))
        lane = jax.lax.broadcasted_iota(jnp.int32, (1, 8), 1)
        tail_ref[...] = jnp.where(lane % 2 == 0,
                                  jnp.broadcast_to(nw, (1, 8)),
                                  jnp.broadcast_to(nb1, (1, 8)))


def _run_layer(x, wk, bias):
    B, N, Fin = x.shape
    K, _, Fout = wk.shape
    return pl.pallas_call(
        functools.partial(_layer_body, K=K, nb=B),
        grid=(B,),
        in_specs=[
            pl.BlockSpec((1, N, Fin), lambda b: (b, 0, 0)),
            pl.BlockSpec((K, Fin, Fout), lambda b: (0, 0, 0)),
            pl.BlockSpec((1, Fout), lambda b: (0, 0)),
        ],
        out_specs=[
            pl.BlockSpec((1, N, Fout), lambda b: (b, 0, 0)),
            pl.BlockSpec((1, 1), lambda b: (0, 0)),
        ],
        out_shape=[
            jax.ShapeDtypeStruct((B, N, Fout), _F32),
            jax.ShapeDtypeStruct((1, 1), _F32),
        ],
        scratch_shapes=[pltpu.VMEM((Fout, Fout), _F32)],
        compiler_params=pltpu.CompilerParams(
            dimension_semantics=("arbitrary",)),
    )(x, wk, bias.reshape(1, Fout))


def _run_layer3_head(x, wk, bias, fc1_w, fc1_b, fc2_w, fc2_b, fc3_w, fc3_b):
    B, N, Fin = x.shape
    K, _, Fout = wk.shape
    ncls = fc3_w.shape[1]
    full = lambda *shape: pl.BlockSpec(shape, lambda b: (0,) * len(shape))
    return pl.pallas_call(
        functools.partial(_layer3_body, K=K, nb=B),
        grid=(B,),
        in_specs=[
            pl.BlockSpec((1, N, Fin), lambda b: (b, 0, 0)),
            full(K, Fin, Fout),
            full(1, Fout),
            full(*fc1_w.shape), full(1, fc1_b.shape[0]),
            full(*fc2_w.shape), full(1, fc2_b.shape[0]),
            full(*fc3_w.shape), full(1, fc3_b.shape[0]),
        ],
        out_specs=[
            full(B, ncls),
            full(1, 1),
            full(1, 8),
        ],
        out_shape=[
            jax.ShapeDtypeStruct((B, ncls), _F32),
            jax.ShapeDtypeStruct((1, 1), _F32),
            jax.ShapeDtypeStruct((1, 8), _F32),
        ],
        scratch_shapes=[pltpu.VMEM((B, Fout), _F32),
                        pltpu.VMEM((Fout, Fout), _F32)],
        compiler_params=pltpu.CompilerParams(
            dimension_semantics=("arbitrary",)),
    )(x, wk, bias.reshape(1, Fout), fc1_w, fc1_b.reshape(1, -1),
      fc2_w, fc2_b.reshape(1, -1), fc3_w, fc3_b.reshape(1, -1))


def kernel(x, conv1_w, conv1_b, conv2_w, conv2_b, conv3_w, conv3_b,
           fc1_w, fc1_b, fc2_w, fc2_b, fc3_w, fc3_b,
           batch, batch_size, nr_points):
    del batch, batch_size, nr_points
    out1, r1 = _run_layer(x, conv1_w, conv1_b)
    out2, r2 = _run_layer(out1, conv2_w, conv2_b)
    logits, r3, tail = _run_layer3_head(
        out2, conv3_w, conv3_b, fc1_w, fc1_b, fc2_w, fc2_b, fc3_w, fc3_b)

    regs = jnp.concatenate([
        r1.reshape(1), r2.reshape(1), r3.reshape(1), tail[0, :6]])
    return logits, regs


# speedup vs baseline: 1.2118x; 1.0214x over previous
import functools

import jax
import jax.numpy as jnp
from jax.experimental import pallas as pl
from jax.experimental.pallas import tpu as pltpu

_F32 = jnp.float32


_BF16 = jnp.bfloat16


def _bdot(a, b, dims):
    return jax.lax.dot_general(a.astype(_BF16), b.astype(_BF16), dims,
                               preferred_element_type=_F32)


_MM = (((1,), (0,)), ((), ()))


def _fdot(a, b, dims):
    return jax.lax.dot_general(a, b, dims, preferred_element_type=_F32)


def _xdot(a, b, dims):
    return jax.lax.dot_general(a, b, dims, precision=jax.lax.Precision.HIGHEST,
                               preferred_element_type=_F32)


def _graph(X):
    N, F = X.shape
    Xsq = X * X
    sq_col = jnp.sum(Xsq, axis=1, keepdims=True)
    ones_row = jnp.ones((1, F), _F32)
    sq_row = _xdot(ones_row, Xsq, (((1,), (1,)), ((), ())))
    X_bf = X.astype(_BF16)
    inner = -2.0 * _fdot(X_bf, X_bf, (((1,), (1,)), ((), ())))
    adj = sq_col + inner + sq_row
    Wg = jnp.exp(-adj)

    rows = jax.lax.broadcasted_iota(jnp.int32, (N, N), 0)
    cols = jax.lax.broadcasted_iota(jnp.int32, (N, N), 1)
    A = jnp.where(rows == cols, 0.0, Wg)

    d_col = jnp.sum(A, axis=1, keepdims=True)
    dinv_col = jnp.where(d_col > 0, 1.0 / jnp.sqrt(jnp.where(d_col > 0, d_col, 1.0)), 0.0)
    dinv_row = jnp.transpose(dinv_col)
    return (A * dinv_col * dinv_row).astype(_BF16)


def _cheb(X, An_bf, wk_ref, bias_ref, K):
    X_bf = X.astype(_BF16)
    wk_bf = [wk_ref[k].astype(_BF16) for k in range(K)]
    out = _fdot(X_bf, wk_bf[0], _MM)
    if K > 1:
        Tx1 = -_fdot(An_bf, X_bf, _MM)
        Tx1_bf = Tx1.astype(_BF16)
        out = out + _fdot(Tx1_bf, wk_bf[1], _MM)
        Tx0 = X
        for k in range(2, K):
            Tx2 = -2.0 * _fdot(An_bf, Tx1_bf, _MM) - Tx0
            Tx2_bf = Tx2.astype(_BF16)
            out = out + _fdot(Tx2_bf, wk_bf[k], _MM)
            Tx0, Tx1_bf = Tx1, Tx2_bf
            Tx1 = Tx2
    return jnp.maximum(out + bias_ref[...], 0.0)


def _mreg_update(out, An_bf, mreg, reg_ref, b, nb):
    out_bf = out.astype(_BF16)
    Lout = out_bf.astype(_F32) - _fdot(An_bf, out_bf, _MM)
    contrib = _fdot(out_bf, Lout.astype(_BF16), (((0,), (0,)), ((), ())))

    @pl.when(b == 0)
    def _():
        mreg[...] = contrib

    @pl.when(b > 0)
    def _():
        mreg[...] = mreg[...] + contrib

    @pl.when(b == nb - 1)
    def _():
        m = mreg[...]
        reg_ref[...] = jnp.broadcast_to(jnp.sqrt(jnp.sum(m * m)), (1, 1))


def _layer_body(x_ref, wk_ref, bias_ref, out_ref, reg_ref, mreg, *, K, nb):
    b = pl.program_id(0)
    X = x_ref[0]
    An_bf = _graph(X)
    out = _cheb(X, An_bf, wk_ref, bias_ref, K)
    out_ref[0] = out
    _mreg_update(out, An_bf, mreg, reg_ref, b, nb)


def _fused_body(x_ref, w1k_ref, b1_ref, w2k_ref, b2_ref, w3k_ref, b3_ref,
                f1w_ref, f1b_ref, f2w_ref, f2b_ref, f3w_ref, f3b_ref,
                logits_ref, r1_ref, r2_ref, r3_ref, tail_ref,
                pooled_scr, mreg1, mreg2, mreg3, *, nb):
    b = pl.program_id(0)
    X = x_ref[0]
    An1 = _graph(X)
    out1 = _cheb(X, An1, w1k_ref, b1_ref, w1k_ref.shape[0])
    _mreg_update(out1, An1, mreg1, r1_ref, b, nb)
    An2 = _graph(out1)
    out2 = _cheb(out1, An2, w2k_ref, b2_ref, w2k_ref.shape[0])
    _mreg_update(out2, An2, mreg2, r2_ref, b, nb)
    An3 = _graph(out2)
    out3 = _cheb(out2, An3, w3k_ref, b3_ref, w3k_ref.shape[0])
    pooled_scr[pl.ds(b, 1), :] = jnp.max(out3, axis=0, keepdims=True)
    _mreg_update(out3, An3, mreg3, r3_ref, b, nb)

    @pl.when(b == nb - 1)
    def _():
        mm = lambda a, w: _bdot(a, w, _MM)
        h = jnp.maximum(mm(pooled_scr[...], f1w_ref[...]) + f1b_ref[...], 0.0)
        h = jnp.maximum(mm(h, f2w_ref[...]) + f2b_ref[...], 0.0)
        logits_ref[...] = mm(h, f3w_ref[...]) + f3b_ref[...]
        w1 = f1w_ref[...]
        nw = jnp.sqrt(jnp.sum(w1 * w1))
        b1 = f1b_ref[...]
        nb1 = jnp.sqrt(jnp.sum(b1 * b1))
        lane = jax.lax.broadcasted_iota(jnp.int32, (1, 8), 1)
        tail_ref[...] = jnp.where(lane % 2 == 0,
                                  jnp.broadcast_to(nw, (1, 8)),
                                  jnp.broadcast_to(nb1, (1, 8)))


def _layer3_body(x_ref, wk_ref, bias_ref,
                 w1_ref, b1_ref, w2_ref, b2_ref, w3_ref, b3_ref,
                 logits_ref, reg_ref, tail_ref, pooled_scr, mreg, *, K, nb):
    b = pl.program_id(0)
    X = x_ref[0]
    An_bf = _graph(X)
    out = _cheb(X, An_bf, wk_ref, bias_ref, K)
    pooled_scr[pl.ds(b, 1), :] = jnp.max(out, axis=0, keepdims=True)
    _mreg_update(out, An_bf, mreg, reg_ref, b, nb)

    @pl.when(b == nb - 1)
    def _():
        mm = lambda a, w: _bdot(a, w, _MM)
        h = jnp.maximum(mm(pooled_scr[...], w1_ref[...]) + b1_ref[...], 0.0)
        h = jnp.maximum(mm(h, w2_ref[...]) + b2_ref[...], 0.0)
        logits_ref[...] = mm(h, w3_ref[...]) + b3_ref[...]
        w1 = w1_ref[...]
        nw = jnp.sqrt(jnp.sum(w1 * w1))
        b1 = b1_ref[...]
        nb1 = jnp.sqrt(jnp.sum(b1 * b1))
        lane = jax.lax.broadcasted_iota(jnp.int32, (1, 8), 1)
        tail_ref[...] = jnp.where(lane % 2 == 0,
                                  jnp.broadcast_to(nw, (1, 8)),
                                  jnp.broadcast_to(nb1, (1, 8)))


def _run_layer(x, wk, bias):
    B, N, Fin = x.shape
    K, _, Fout = wk.shape
    return pl.pallas_call(
        functools.partial(_layer_body, K=K, nb=B),
        grid=(B,),
        in_specs=[
            pl.BlockSpec((1, N, Fin), lambda b: (b, 0, 0)),
            pl.BlockSpec((K, Fin, Fout), lambda b: (0, 0, 0)),
            pl.BlockSpec((1, Fout), lambda b: (0, 0)),
        ],
        out_specs=[
            pl.BlockSpec((1, N, Fout), lambda b: (b, 0, 0)),
            pl.BlockSpec((1, 1), lambda b: (0, 0)),
        ],
        out_shape=[
            jax.ShapeDtypeStruct((B, N, Fout), _F32),
            jax.ShapeDtypeStruct((1, 1), _F32),
        ],
        scratch_shapes=[pltpu.VMEM((Fout, Fout), _F32)],
        compiler_params=pltpu.CompilerParams(
            dimension_semantics=("arbitrary",)),
    )(x, wk, bias.reshape(1, Fout))


def _run_layer3_head(x, wk, bias, fc1_w, fc1_b, fc2_w, fc2_b, fc3_w, fc3_b):
    B, N, Fin = x.shape
    K, _, Fout = wk.shape
    ncls = fc3_w.shape[1]
    full = lambda *shape: pl.BlockSpec(shape, lambda b: (0,) * len(shape))
    return pl.pallas_call(
        functools.partial(_layer3_body, K=K, nb=B),
        grid=(B,),
        in_specs=[
            pl.BlockSpec((1, N, Fin), lambda b: (b, 0, 0)),
            full(K, Fin, Fout),
            full(1, Fout),
            full(*fc1_w.shape), full(1, fc1_b.shape[0]),
            full(*fc2_w.shape), full(1, fc2_b.shape[0]),
            full(*fc3_w.shape), full(1, fc3_b.shape[0]),
        ],
        out_specs=[
            full(B, ncls),
            full(1, 1),
            full(1, 8),
        ],
        out_shape=[
            jax.ShapeDtypeStruct((B, ncls), _F32),
            jax.ShapeDtypeStruct((1, 1), _F32),
            jax.ShapeDtypeStruct((1, 8), _F32),
        ],
        scratch_shapes=[pltpu.VMEM((B, Fout), _F32),
                        pltpu.VMEM((Fout, Fout), _F32)],
        compiler_params=pltpu.CompilerParams(
            dimension_semantics=("arbitrary",)),
    )(x, wk, bias.reshape(1, Fout), fc1_w, fc1_b.reshape(1, -1),
      fc2_w, fc2_b.reshape(1, -1), fc3_w, fc3_b.reshape(1, -1))


def kernel(x, conv1_w, conv1_b, conv2_w, conv2_b, conv3_w, conv3_b,
           fc1_w, fc1_b, fc2_w, fc2_b, fc3_w, fc3_b,
           batch, batch_size, nr_points):
    del batch, batch_size, nr_points
    B, N, _ = x.shape
    F1 = conv1_w.shape[2]
    F2 = conv2_w.shape[2]
    F3 = conv3_w.shape[2]
    ncls = fc3_w.shape[1]
    full = lambda *shape: pl.BlockSpec(shape, lambda b: (0,) * len(shape))
    logits, r1, r2, r3, tail = pl.pallas_call(
        functools.partial(_fused_body, nb=B),
        grid=(B,),
        in_specs=[
            pl.BlockSpec((1, N, x.shape[2]), lambda b: (b, 0, 0)),
            full(*conv1_w.shape), full(1, F1),
            full(*conv2_w.shape), full(1, F2),
            full(*conv3_w.shape), full(1, F3),
            full(*fc1_w.shape), full(1, fc1_b.shape[0]),
            full(*fc2_w.shape), full(1, fc2_b.shape[0]),
            full(*fc3_w.shape), full(1, fc3_b.shape[0]),
        ],
        out_specs=[full(B, ncls), full(1, 1), full(1, 1), full(1, 1),
                   full(1, 8)],
        out_shape=[
            jax.ShapeDtypeStruct((B, ncls), _F32),
            jax.ShapeDtypeStruct((1, 1), _F32),
            jax.ShapeDtypeStruct((1, 1), _F32),
            jax.ShapeDtypeStruct((1, 1), _F32),
            jax.ShapeDtypeStruct((1, 8), _F32),
        ],
        scratch_shapes=[pltpu.VMEM((B, F3), _F32),
                        pltpu.VMEM((F1, F1), _F32),
                        pltpu.VMEM((F2, F2), _F32),
                        pltpu.VMEM((F3, F3), _F32)],
        compiler_params=pltpu.CompilerParams(
            dimension_semantics=("arbitrary",)),
    )(x, conv1_w, conv1_b.reshape(1, F1), conv2_w, conv2_b.reshape(1, F2),
      conv3_w, conv3_b.reshape(1, F3), fc1_w, fc1_b.reshape(1, -1),
      fc2_w, fc2_b.reshape(1, -1), fc3_w, fc3_b.reshape(1, -1))

    regs = jnp.concatenate([
        r1.reshape(1), r2.reshape(1), r3.reshape(1), tail[0, :6]])
    return logits, regs


# speedup vs baseline: 1.2275x; 1.0129x over previous
import functools

import jax
import jax.numpy as jnp
from jax.experimental import pallas as pl
from jax.experimental.pallas import tpu as pltpu

_F32 = jnp.float32


_BF16 = jnp.bfloat16


def _bdot(a, b, dims):
    return jax.lax.dot_general(a.astype(_BF16), b.astype(_BF16), dims,
                               preferred_element_type=_F32)


_MM = (((1,), (0,)), ((), ()))


def _fdot(a, b, dims):
    return jax.lax.dot_general(a, b, dims, preferred_element_type=_F32)


def _xdot(a, b, dims):
    return jax.lax.dot_general(a, b, dims, precision=jax.lax.Precision.HIGHEST,
                               preferred_element_type=_F32)


def _graph(X):
    N, F = X.shape
    Xsq = X * X
    sq_col = jnp.sum(Xsq, axis=1, keepdims=True)
    ones_row = jnp.ones((1, F), _F32)
    sq_row = _xdot(ones_row, Xsq, (((1,), (1,)), ((), ())))
    X_bf = X.astype(_BF16)
    inner = -2.0 * _fdot(X_bf, X_bf, (((1,), (1,)), ((), ())))
    adj = sq_col + inner + sq_row
    Wg = jnp.exp(-adj)

    rows = jax.lax.broadcasted_iota(jnp.int32, (N, N), 0)
    cols = jax.lax.broadcasted_iota(jnp.int32, (N, N), 1)
    A = jnp.where(rows == cols, 0.0, Wg)

    d_col = jnp.sum(A, axis=1, keepdims=True)
    dinv_col = jnp.where(d_col > 0, 1.0 / jnp.sqrt(jnp.where(d_col > 0, d_col, 1.0)), 0.0)
    dinv_row = jnp.transpose(dinv_col)
    return (A * dinv_col * dinv_row).astype(_BF16)


def _cheb(X, An_bf, wk_ref, bias_ref, K):
    X_bf = X.astype(_BF16)
    wk_bf = [wk_ref[k].astype(_BF16) for k in range(K)]
    out = _fdot(X_bf, wk_bf[0], _MM)
    if K > 1:
        Tx1 = -_fdot(An_bf, X_bf, _MM)
        Tx1_bf = Tx1.astype(_BF16)
        out = out + _fdot(Tx1_bf, wk_bf[1], _MM)
        Tx0 = X
        for k in range(2, K):
            Tx2 = -2.0 * _fdot(An_bf, Tx1_bf, _MM) - Tx0
            Tx2_bf = Tx2.astype(_BF16)
            out = out + _fdot(Tx2_bf, wk_bf[k], _MM)
            Tx0, Tx1_bf = Tx1, Tx2_bf
            Tx1 = Tx2
    return jnp.maximum(out + bias_ref[...], 0.0)


def _mreg_update(out, An_bf, mreg, reg_ref, b, nb):
    out_bf = out.astype(_BF16)
    Lout = out_bf.astype(_F32) - _fdot(An_bf, out_bf, _MM)
    contrib = _fdot(out_bf, Lout.astype(_BF16), (((0,), (0,)), ((), ())))

    @pl.when(b == 0)
    def _():
        mreg[...] = contrib

    @pl.when(b > 0)
    def _():
        mreg[...] = mreg[...] + contrib

    @pl.when(b == nb - 1)
    def _():
        m = mreg[...]
        reg_ref[...] = jnp.broadcast_to(jnp.sqrt(jnp.sum(m * m)), (1, 1))


def _graph_mreg(X, An_prev_bf, mreg, reg_ref, b, nb, nchunk=4):
    N, F = X.shape
    C = N // nchunk
    Xsq = X * X
    sq_col = jnp.sum(Xsq, axis=1, keepdims=True)
    ones_row = jnp.ones((1, F), _F32)
    sq_row = _xdot(ones_row, Xsq, (((1,), (1,)), ((), ())))
    X_bf = X.astype(_BF16)

    a_parts, contrib = [], None
    for c in range(nchunk):
        r0 = c * C
        inner_c = -2.0 * _fdot(X_bf[r0:r0 + C], X_bf, (((1,), (1,)), ((), ())))
        adj_c = sq_col[r0:r0 + C] + inner_c + sq_row
        Wg_c = jnp.exp(-adj_c)
        rows_c = jax.lax.broadcasted_iota(jnp.int32, (C, N), 0) + r0
        cols_c = jax.lax.broadcasted_iota(jnp.int32, (C, N), 1)
        a_parts.append(jnp.where(rows_c == cols_c, 0.0, Wg_c))

        ob_c = X_bf[r0:r0 + C]
        Lout_c = ob_c.astype(_F32) - _fdot(An_prev_bf[r0:r0 + C], X_bf, _MM)
        part = _fdot(ob_c, Lout_c.astype(_BF16), (((0,), (0,)), ((), ())))
        contrib = part if contrib is None else contrib + part

    A = jnp.concatenate(a_parts, axis=0)
    d_col = jnp.sum(A, axis=1, keepdims=True)
    dinv_col = jnp.where(d_col > 0, 1.0 / jnp.sqrt(jnp.where(d_col > 0, d_col, 1.0)), 0.0)
    dinv_row = jnp.transpose(dinv_col)
    An_bf = (A * dinv_col * dinv_row).astype(_BF16)

    @pl.when(b == 0)
    def _():
        mreg[...] = contrib

    @pl.when(b > 0)
    def _():
        mreg[...] = mreg[...] + contrib

    @pl.when(b == nb - 1)
    def _():
        m = mreg[...]
        reg_ref[...] = jnp.broadcast_to(jnp.sqrt(jnp.sum(m * m)), (1, 1))

    return An_bf


def _layer_body(x_ref, wk_ref, bias_ref, out_ref, reg_ref, mreg, *, K, nb):
    b = pl.program_id(0)
    X = x_ref[0]
    An_bf = _graph(X)
    out = _cheb(X, An_bf, wk_ref, bias_ref, K)
    out_ref[0] = out
    _mreg_update(out, An_bf, mreg, reg_ref, b, nb)


def _fused_body(x_ref, w1k_ref, b1_ref, w2k_ref, b2_ref, w3k_ref, b3_ref,
                f1w_ref, f1b_ref, f2w_ref, f2b_ref, f3w_ref, f3b_ref,
                logits_ref, r1_ref, r2_ref, r3_ref, tail_ref,
                pooled_scr, mreg1, mreg2, mreg3, *, nb):
    b = pl.program_id(0)
    X = x_ref[0]
    An1 = _graph(X)
    out1 = _cheb(X, An1, w1k_ref, b1_ref, w1k_ref.shape[0])
    An2 = _graph_mreg(out1, An1, mreg1, r1_ref, b, nb)
    out2 = _cheb(out1, An2, w2k_ref, b2_ref, w2k_ref.shape[0])
    An3 = _graph_mreg(out2, An2, mreg2, r2_ref, b, nb)
    out3 = _cheb(out2, An3, w3k_ref, b3_ref, w3k_ref.shape[0])
    pooled_scr[pl.ds(b, 1), :] = jnp.max(out3, axis=0, keepdims=True)
    _mreg_update(out3, An3, mreg3, r3_ref, b, nb)

    @pl.when(b == nb - 1)
    def _():
        mm = lambda a, w: _bdot(a, w, _MM)
        h = jnp.maximum(mm(pooled_scr[...], f1w_ref[...]) + f1b_ref[...], 0.0)
        h = jnp.maximum(mm(h, f2w_ref[...]) + f2b_ref[...], 0.0)
        logits_ref[...] = mm(h, f3w_ref[...]) + f3b_ref[...]
        w1 = f1w_ref[...]
        nw = jnp.sqrt(jnp.sum(w1 * w1))
        b1 = f1b_ref[...]
        nb1 = jnp.sqrt(jnp.sum(b1 * b1))
        lane = jax.lax.broadcasted_iota(jnp.int32, (1, 8), 1)
        tail_ref[...] = jnp.where(lane % 2 == 0,
                                  jnp.broadcast_to(nw, (1, 8)),
                                  jnp.broadcast_to(nb1, (1, 8)))


def _layer3_body(x_ref, wk_ref, bias_ref,
                 w1_ref, b1_ref, w2_ref, b2_ref, w3_ref, b3_ref,
                 logits_ref, reg_ref, tail_ref, pooled_scr, mreg, *, K, nb):
    b = pl.program_id(0)
    X = x_ref[0]
    An_bf = _graph(X)
    out = _cheb(X, An_bf, wk_ref, bias_ref, K)
    pooled_scr[pl.ds(b, 1), :] = jnp.max(out, axis=0, keepdims=True)
    _mreg_update(out, An_bf, mreg, reg_ref, b, nb)

    @pl.when(b == nb - 1)
    def _():
        mm = lambda a, w: _bdot(a, w, _MM)
        h = jnp.maximum(mm(pooled_scr[...], w1_ref[...]) + b1_ref[...], 0.0)
        h = jnp.maximum(mm(h, w2_ref[...]) + b2_ref[...], 0.0)
        logits_ref[...] = mm(h, w3_ref[...]) + b3_ref[...]
        w1 = w1_ref[...]
        nw = jnp.sqrt(jnp.sum(w1 * w1))
        b1 = b1_ref[...]
        nb1 = jnp.sqrt(jnp.sum(b1 * b1))
        lane = jax.lax.broadcasted_iota(jnp.int32, (1, 8), 1)
        tail_ref[...] = jnp.where(lane % 2 == 0,
                                  jnp.broadcast_to(nw, (1, 8)),
                                  jnp.broadcast_to(nb1, (1, 8)))


def _run_layer(x, wk, bias):
    B, N, Fin = x.shape
    K, _, Fout = wk.shape
    return pl.pallas_call(
        functools.partial(_layer_body, K=K, nb=B),
        grid=(B,),
        in_specs=[
            pl.BlockSpec((1, N, Fin), lambda b: (b, 0, 0)),
            pl.BlockSpec((K, Fin, Fout), lambda b: (0, 0, 0)),
            pl.BlockSpec((1, Fout), lambda b: (0, 0)),
        ],
        out_specs=[
            pl.BlockSpec((1, N, Fout), lambda b: (b, 0, 0)),
            pl.BlockSpec((1, 1), lambda b: (0, 0)),
        ],
        out_shape=[
            jax.ShapeDtypeStruct((B, N, Fout), _F32),
            jax.ShapeDtypeStruct((1, 1), _F32),
        ],
        scratch_shapes=[pltpu.VMEM((Fout, Fout), _F32)],
        compiler_params=pltpu.CompilerParams(
            dimension_semantics=("arbitrary",)),
    )(x, wk, bias.reshape(1, Fout))


def _run_layer3_head(x, wk, bias, fc1_w, fc1_b, fc2_w, fc2_b, fc3_w, fc3_b):
    B, N, Fin = x.shape
    K, _, Fout = wk.shape
    ncls = fc3_w.shape[1]
    full = lambda *shape: pl.BlockSpec(shape, lambda b: (0,) * len(shape))
    return pl.pallas_call(
        functools.partial(_layer3_body, K=K, nb=B),
        grid=(B,),
        in_specs=[
            pl.BlockSpec((1, N, Fin), lambda b: (b, 0, 0)),
            full(K, Fin, Fout),
            full(1, Fout),
            full(*fc1_w.shape), full(1, fc1_b.shape[0]),
            full(*fc2_w.shape), full(1, fc2_b.shape[0]),
            full(*fc3_w.shape), full(1, fc3_b.shape[0]),
        ],
        out_specs=[
            full(B, ncls),
            full(1, 1),
            full(1, 8),
        ],
        out_shape=[
            jax.ShapeDtypeStruct((B, ncls), _F32),
            jax.ShapeDtypeStruct((1, 1), _F32),
            jax.ShapeDtypeStruct((1, 8), _F32),
        ],
        scratch_shapes=[pltpu.VMEM((B, Fout), _F32),
                        pltpu.VMEM((Fout, Fout), _F32)],
        compiler_params=pltpu.CompilerParams(
            dimension_semantics=("arbitrary",)),
    )(x, wk, bias.reshape(1, Fout), fc1_w, fc1_b.reshape(1, -1),
      fc2_w, fc2_b.reshape(1, -1), fc3_w, fc3_b.reshape(1, -1))


def kernel(x, conv1_w, conv1_b, conv2_w, conv2_b, conv3_w, conv3_b,
           fc1_w, fc1_b, fc2_w, fc2_b, fc3_w, fc3_b,
           batch, batch_size, nr_points):
    del batch, batch_size, nr_points
    B, N, _ = x.shape
    F1 = conv1_w.shape[2]
    F2 = conv2_w.shape[2]
    F3 = conv3_w.shape[2]
    ncls = fc3_w.shape[1]
    full = lambda *shape: pl.BlockSpec(shape, lambda b: (0,) * len(shape))
    logits, r1, r2, r3, tail = pl.pallas_call(
        functools.partial(_fused_body, nb=B),
        grid=(B,),
        in_specs=[
            pl.BlockSpec((1, N, x.shape[2]), lambda b: (b, 0, 0)),
            full(*conv1_w.shape), full(1, F1),
            full(*conv2_w.shape), full(1, F2),
            full(*conv3_w.shape), full(1, F3),
            full(*fc1_w.shape), full(1, fc1_b.shape[0]),
            full(*fc2_w.shape), full(1, fc2_b.shape[0]),
            full(*fc3_w.shape), full(1, fc3_b.shape[0]),
        ],
        out_specs=[full(B, ncls), full(1, 1), full(1, 1), full(1, 1),
                   full(1, 8)],
        out_shape=[
            jax.ShapeDtypeStruct((B, ncls), _F32),
            jax.ShapeDtypeStruct((1, 1), _F32),
            jax.ShapeDtypeStruct((1, 1), _F32),
            jax.ShapeDtypeStruct((1, 1), _F32),
            jax.ShapeDtypeStruct((1, 8), _F32),
        ],
        scratch_shapes=[pltpu.VMEM((B, F3), _F32),
                        pltpu.VMEM((F1, F1), _F32),
                        pltpu.VMEM((F2, F2), _F32),
                        pltpu.VMEM((F3, F3), _F32)],
        compiler_params=pltpu.CompilerParams(
            dimension_semantics=("arbitrary",)),
    )(x, conv1_w, conv1_b.reshape(1, F1), conv2_w, conv2_b.reshape(1, F2),
      conv3_w, conv3_b.reshape(1, F3), fc1_w, fc1_b.reshape(1, -1),
      fc2_w, fc2_b.reshape(1, -1), fc3_w, fc3_b.reshape(1, -1))

    regs = jnp.concatenate([
        r1.reshape(1), r2.reshape(1), r3.reshape(1), tail[0, :6]])
    return logits, regs


# speedup vs baseline: 1.2450x; 1.0143x over previous
import functools

import jax
import jax.numpy as jnp
from jax.experimental import pallas as pl
from jax.experimental.pallas import tpu as pltpu

_F32 = jnp.float32
_BF16 = jnp.bfloat16
_MM = (((1,), (0,)), ((), ()))
_NCHUNK = 4


def _bdot(a, b, dims):
    return jax.lax.dot_general(a.astype(_BF16), b.astype(_BF16), dims,
                               preferred_element_type=_F32)


def _fdot(a, b, dims):
    return jax.lax.dot_general(a, b, dims, preferred_element_type=_F32)


def _xdot(a, b, dims):
    return jax.lax.dot_general(a, b, dims, precision=jax.lax.Precision.HIGHEST,
                               preferred_element_type=_F32)


def _mreg_emit(contrib, mreg, b):
    @pl.when(b == 0)
    def _():
        mreg[...] = contrib

    @pl.when(b > 0)
    def _():
        mreg[...] = mreg[...] + contrib


def _fro(m):
    return jnp.sqrt(jnp.sum(m * m))


def _graph_chunks(X, mreg_chunk_fn=None):
    N, F = X.shape
    C = N // _NCHUNK
    Xsq = X * X
    sq_col = jnp.sum(Xsq, axis=1, keepdims=True)
    ones_row = jnp.ones((1, F), _F32)
    sq_row = _xdot(ones_row, Xsq, (((1,), (1,)), ((), ())))
    X_bf = X.astype(_BF16)

    a_parts = []
    mreg_parts = []
    for c in range(_NCHUNK):
        r0 = c * C
        inner_c = -2.0 * _fdot(X_bf[r0:r0 + C], X_bf, (((1,), (1,)), ((), ())))
        adj_c = sq_col[r0:r0 + C] + inner_c + sq_row
        Wg_c = jnp.exp(-adj_c)
        rows_c = jax.lax.broadcasted_iota(jnp.int32, (C, N), 0) + r0
        cols_c = jax.lax.broadcasted_iota(jnp.int32, (C, N), 1)
        a_parts.append(jnp.where(rows_c == cols_c, 0.0, Wg_c))
        if mreg_chunk_fn is not None:
            mreg_parts.append(mreg_chunk_fn(c))

    d_col = jnp.concatenate(
        [jnp.sum(a, axis=1, keepdims=True) for a in a_parts], axis=0)
    dinv_col = jnp.where(d_col > 0,
                         1.0 / jnp.sqrt(jnp.where(d_col > 0, d_col, 1.0)), 0.0)
    dinv_row = jnp.transpose(dinv_col)
    an_parts = [
        (a_parts[c] * dinv_col[c * C:(c + 1) * C] * dinv_row).astype(_BF16)
        for c in range(_NCHUNK)]
    if mreg_chunk_fn is not None:
        return an_parts, sum(mreg_parts[1:], mreg_parts[0])
    return an_parts


def _mreg_chunk(an_parts, out_bf, c):
    N = out_bf.shape[0]
    C = N // _NCHUNK
    ob_c = out_bf[c * C:(c + 1) * C]
    Lout_c = ob_c.astype(_F32) - _fdot(an_parts[c], out_bf, _MM)
    return _fdot(ob_c, Lout_c.astype(_BF16), (((0,), (0,)), ((), ())))


def _cheb(X, an_parts, wk_ref, bias_ref, K):
    N = X.shape[0]
    C = N // _NCHUNK
    rs = [slice(c * C, (c + 1) * C) for c in range(_NCHUNK)]
    X_bf = X.astype(_BF16)
    wk_bf = [wk_ref[k].astype(_BF16) for k in range(K)]
    out_p = [_fdot(X_bf[r], wk_bf[0], _MM) for r in rs]
    if K > 1:
        Tx1_p, Tx1b_p = [], []
        for c, r in enumerate(rs):
            t1 = -_fdot(an_parts[c], X_bf, _MM)
            t1b = t1.astype(_BF16)
            out_p[c] = out_p[c] + _fdot(t1b, wk_bf[1], _MM)
            Tx1_p.append(t1)
            Tx1b_p.append(t1b)
        Tx0_p = [X[r] for r in rs]
        Tx1_bf = jnp.concatenate(Tx1b_p, axis=0)
        for k in range(2, K):
            Tx2_p, Tx2b_p = [], []
            for c in range(_NCHUNK):
                t2 = -2.0 * _fdot(an_parts[c], Tx1_bf, _MM) - Tx0_p[c]
                t2b = t2.astype(_BF16)
                out_p[c] = out_p[c] + _fdot(t2b, wk_bf[k], _MM)
                Tx2_p.append(t2)
                Tx2b_p.append(t2b)
            Tx0_p, Tx1_p = Tx1_p, Tx2_p
            Tx1_bf = jnp.concatenate(Tx2b_p, axis=0)
    return jnp.concatenate(
        [jnp.maximum(o + bias_ref[...], 0.0) for o in out_p], axis=0)


def _fused_body(x_ref, w1k_ref, b1_ref, w2k_ref, b2_ref, w3k_ref, b3_ref,
                f1w_ref, f1b_ref, f2w_ref, f2b_ref, f3w_ref, f3b_ref,
                logits_ref, regs_ref,
                pooled_scr, mreg1, mreg2, mreg3, *, nb):
    b = pl.program_id(0)
    X = x_ref[0]
    an1 = _graph_chunks(X)
    out1 = _cheb(X, an1, w1k_ref, b1_ref, w1k_ref.shape[0])

    out1_bf = out1.astype(_BF16)
    an2, contrib1 = _graph_chunks(
        out1, lambda c: _mreg_chunk(an1, out1_bf, c))
    _mreg_emit(contrib1, mreg1, b)
    out2 = _cheb(out1, an2, w2k_ref, b2_ref, w2k_ref.shape[0])

    out2_bf = out2.astype(_BF16)
    an3, contrib2 = _graph_chunks(
        out2, lambda c: _mreg_chunk(an2, out2_bf, c))
    _mreg_emit(contrib2, mreg2, b)
    out3 = _cheb(out2, an3, w3k_ref, b3_ref, w3k_ref.shape[0])

    pooled_scr[pl.ds(b, 1), :] = jnp.max(out3, axis=0, keepdims=True)
    out3_bf = out3.astype(_BF16)
    parts3 = [_mreg_chunk(an3, out3_bf, c) for c in range(_NCHUNK)]
    _mreg_emit(sum(parts3[1:], parts3[0]), mreg3, b)

    @pl.when(b == nb - 1)
    def _():
        mm = lambda a, w: _bdot(a, w, _MM)
        h = jnp.maximum(mm(pooled_scr[...], f1w_ref[...]) + f1b_ref[...], 0.0)
        h = jnp.maximum(mm(h, f2w_ref[...]) + f2b_ref[...], 0.0)
        logits_ref[...] = mm(h, f3w_ref[...]) + f3b_ref[...]
        w1 = f1w_ref[...]
        nw = jnp.sqrt(jnp.sum(w1 * w1))
        b1 = f1b_ref[...]
        nb1 = jnp.sqrt(jnp.sum(b1 * b1))
        lane = jax.lax.broadcasted_iota(jnp.int32, (1, 9), 1)
        bc = lambda v: jnp.broadcast_to(v, (1, 9))
        regs_ref[...] = jnp.where(
            lane == 0, bc(_fro(mreg1[...])),
            jnp.where(lane == 1, bc(_fro(mreg2[...])),
                      jnp.where(lane == 2, bc(_fro(mreg3[...])),
                                jnp.where((lane - 3) % 2 == 0, bc(nw),
                                          bc(nb1)))))


def kernel(x, conv1_w, conv1_b, conv2_w, conv2_b, conv3_w, conv3_b,
           fc1_w, fc1_b, fc2_w, fc2_b, fc3_w, fc3_b,
           batch, batch_size, nr_points):
    del batch, batch_size, nr_points
    B, N, _ = x.shape
    F1 = conv1_w.shape[2]
    F2 = conv2_w.shape[2]
    F3 = conv3_w.shape[2]
    ncls = fc3_w.shape[1]
    full = lambda *shape: pl.BlockSpec(shape, lambda b: (0,) * len(shape))
    logits, regs = pl.pallas_call(
        functools.partial(_fused_body, nb=B),
        grid=(B,),
        in_specs=[
            pl.BlockSpec((1, N, x.shape[2]), lambda b: (b, 0, 0)),
            full(*conv1_w.shape), full(1, F1),
            full(*conv2_w.shape), full(1, F2),
            full(*conv3_w.shape), full(1, F3),
            full(*fc1_w.shape), full(1, fc1_b.shape[0]),
            full(*fc2_w.shape), full(1, fc2_b.shape[0]),
            full(*fc3_w.shape), full(1, fc3_b.shape[0]),
        ],
        out_specs=[full(B, ncls), full(1, 9)],
        out_shape=[
            jax.ShapeDtypeStruct((B, ncls), _F32),
            jax.ShapeDtypeStruct((1, 9), _F32),
        ],
        scratch_shapes=[pltpu.VMEM((B, F3), _F32),
                        pltpu.VMEM((F1, F1), _F32),
                        pltpu.VMEM((F2, F2), _F32),
                        pltpu.VMEM((F3, F3), _F32)],
        compiler_params=pltpu.CompilerParams(
            dimension_semantics=("arbitrary",)),
    )(x, conv1_w, conv1_b.reshape(1, F1), conv2_w, conv2_b.reshape(1, F2),
      conv3_w, conv3_b.reshape(1, F3), fc1_w, fc1_b.reshape(1, -1),
      fc2_w, fc2_b.reshape(1, -1), fc3_w, fc3_b.reshape(1, -1))

    return logits, regs.reshape(9)


# speedup vs baseline: 1.2465x; 1.0012x over previous
import functools

import jax
import jax.numpy as jnp
from jax.experimental import pallas as pl
from jax.experimental.pallas import tpu as pltpu

_F32 = jnp.float32
_BF16 = jnp.bfloat16
_MM = (((1,), (0,)), ((), ()))
_NCHUNK = 4


def _bdot(a, b, dims):
    return jax.lax.dot_general(a.astype(_BF16), b.astype(_BF16), dims,
                               preferred_element_type=_F32)


def _fdot(a, b, dims):
    return jax.lax.dot_general(a, b, dims, preferred_element_type=_F32)


def _xdot(a, b, dims):
    return jax.lax.dot_general(a, b, dims, precision=jax.lax.Precision.HIGHEST,
                               preferred_element_type=_F32)


def _mreg_emit(contrib, mreg, b):
    @pl.when(b == 0)
    def _():
        mreg[...] = contrib

    @pl.when(b > 0)
    def _():
        mreg[...] = mreg[...] + contrib


def _fro(m):
    return jnp.sqrt(jnp.sum(m * m))


def _graph_chunks(X, mreg_chunk_fn=None):
    N, F = X.shape
    C = N // _NCHUNK
    Xsq = X * X
    sq_col = jnp.sum(Xsq, axis=1, keepdims=True)
    ones_row = jnp.ones((1, F), _F32)
    sq_row = _xdot(ones_row, Xsq, (((1,), (1,)), ((), ())))
    X_bf = X.astype(_BF16)

    a_parts = []
    d_parts = []
    mreg_parts = []
    for c in range(_NCHUNK):
        r0 = c * C
        inner_c = -2.0 * _fdot(X_bf[r0:r0 + C], X_bf, (((1,), (1,)), ((), ())))
        adj_c = sq_col[r0:r0 + C] + inner_c + sq_row
        Wg_c = jnp.exp(-adj_c)
        rows_c = jax.lax.broadcasted_iota(jnp.int32, (C, N), 0) + r0
        cols_c = jax.lax.broadcasted_iota(jnp.int32, (C, N), 1)
        a_c = jnp.where(rows_c == cols_c, 0.0, Wg_c)
        a_parts.append(a_c)
        d_parts.append(jnp.sum(a_c, axis=1, keepdims=True))
        if mreg_chunk_fn is not None:
            mreg_parts.append(mreg_chunk_fn(c))

    d_col = jnp.concatenate(d_parts, axis=0)
    dinv_col = jnp.where(d_col > 0,
                         1.0 / jnp.sqrt(jnp.where(d_col > 0, d_col, 1.0)), 0.0)
    dinv_row = jnp.transpose(dinv_col)
    an_parts = [
        (a_parts[c] * dinv_col[c * C:(c + 1) * C] * dinv_row).astype(_BF16)
        for c in range(_NCHUNK)]
    if mreg_chunk_fn is not None:
        return an_parts, sum(mreg_parts[1:], mreg_parts[0])
    return an_parts


def _mreg_chunk(an_parts, out_bf, c):
    N = out_bf.shape[0]
    C = N // _NCHUNK
    ob_c = out_bf[c * C:(c + 1) * C]
    Lout_c = ob_c.astype(_F32) - _fdot(an_parts[c], out_bf, _MM)
    return _fdot(ob_c, Lout_c.astype(_BF16), (((0,), (0,)), ((), ())))


def _cheb(X, an_parts, wk_ref, bias_ref, K):
    N = X.shape[0]
    C = N // _NCHUNK
    rs = [slice(c * C, (c + 1) * C) for c in range(_NCHUNK)]
    X_bf = X.astype(_BF16)
    wk_bf = [wk_ref[k].astype(_BF16) for k in range(K)]
    out_p = [_fdot(X_bf[r], wk_bf[0], _MM) for r in rs]
    if K > 1:
        Tx1_p, Tx1b_p = [], []
        for c, r in enumerate(rs):
            t1 = -_fdot(an_parts[c], X_bf, _MM)
            t1b = t1.astype(_BF16)
            out_p[c] = out_p[c] + _fdot(t1b, wk_bf[1], _MM)
            Tx1_p.append(t1)
            Tx1b_p.append(t1b)
        Tx0_p = [X[r] for r in rs]
        Tx1_bf = jnp.concatenate(Tx1b_p, axis=0)
        for k in range(2, K):
            Tx2_p, Tx2b_p = [], []
            for c in range(_NCHUNK):
                t2 = -2.0 * _fdot(an_parts[c], Tx1_bf, _MM) - Tx0_p[c]
                t2b = t2.astype(_BF16)
                out_p[c] = out_p[c] + _fdot(t2b, wk_bf[k], _MM)
                Tx2_p.append(t2)
                Tx2b_p.append(t2b)
            Tx0_p, Tx1_p = Tx1_p, Tx2_p
            Tx1_bf = jnp.concatenate(Tx2b_p, axis=0)
    return jnp.concatenate(
        [jnp.maximum(o + bias_ref[...], 0.0) for o in out_p], axis=0)


def _fused_body(x_ref, w1k_ref, b1_ref, w2k_ref, b2_ref, w3k_ref, b3_ref,
                f1w_ref, f1b_ref, f2w_ref, f2b_ref, f3w_ref, f3b_ref,
                logits_ref, regs_ref,
                pooled_scr, mreg1, mreg2, mreg3, *, nb):
    b = pl.program_id(0)
    X = x_ref[0]
    an1 = _graph_chunks(X)
    out1 = _cheb(X, an1, w1k_ref, b1_ref, w1k_ref.shape[0])

    out1_bf = out1.astype(_BF16)
    an2, contrib1 = _graph_chunks(
        out1, lambda c: _mreg_chunk(an1, out1_bf, c))
    _mreg_emit(contrib1, mreg1, b)
    out2 = _cheb(out1, an2, w2k_ref, b2_ref, w2k_ref.shape[0])

    out2_bf = out2.astype(_BF16)
    an3, contrib2 = _graph_chunks(
        out2, lambda c: _mreg_chunk(an2, out2_bf, c))
    _mreg_emit(contrib2, mreg2, b)
    out3 = _cheb(out2, an3, w3k_ref, b3_ref, w3k_ref.shape[0])

    pooled_scr[pl.ds(b, 1), :] = jnp.max(out3, axis=0, keepdims=True)
    out3_bf = out3.astype(_BF16)
    parts3 = [_mreg_chunk(an3, out3_bf, c) for c in range(_NCHUNK)]
    _mreg_emit(sum(parts3[1:], parts3[0]), mreg3, b)

    @pl.when(b == nb - 1)
    def _():
        mm = lambda a, w: _bdot(a, w, _MM)
        h = jnp.maximum(mm(pooled_scr[...], f1w_ref[...]) + f1b_ref[...], 0.0)
        h = jnp.maximum(mm(h, f2w_ref[...]) + f2b_ref[...], 0.0)
        logits_ref[...] = mm(h, f3w_ref[...]) + f3b_ref[...]
        w1 = f1w_ref[...]
        nw = jnp.sqrt(jnp.sum(w1 * w1))
        b1 = f1b_ref[...]
        nb1 = jnp.sqrt(jnp.sum(b1 * b1))
        lane = jax.lax.broadcasted_iota(jnp.int32, (1, 9), 1)
        bc = lambda v: jnp.broadcast_to(v, (1, 9))
        regs_ref[...] = jnp.where(
            lane == 0, bc(_fro(mreg1[...])),
            jnp.where(lane == 1, bc(_fro(mreg2[...])),
                      jnp.where(lane == 2, bc(_fro(mreg3[...])),
                                jnp.where((lane - 3) % 2 == 0, bc(nw),
                                          bc(nb1)))))


def kernel(x, conv1_w, conv1_b, conv2_w, conv2_b, conv3_w, conv3_b,
           fc1_w, fc1_b, fc2_w, fc2_b, fc3_w, fc3_b,
           batch, batch_size, nr_points):
    del batch, batch_size, nr_points
    B, N, _ = x.shape
    F1 = conv1_w.shape[2]
    F2 = conv2_w.shape[2]
    F3 = conv3_w.shape[2]
    ncls = fc3_w.shape[1]
    full = lambda *shape: pl.BlockSpec(shape, lambda b: (0,) * len(shape))
    logits, regs = pl.pallas_call(
        functools.partial(_fused_body, nb=B),
        grid=(B,),
        in_specs=[
            pl.BlockSpec((1, N, x.shape[2]), lambda b: (b, 0, 0)),
            full(*conv1_w.shape), full(1, F1),
            full(*conv2_w.shape), full(1, F2),
            full(*conv3_w.shape), full(1, F3),
            full(*fc1_w.shape), full(1, fc1_b.shape[0]),
            full(*fc2_w.shape), full(1, fc2_b.shape[0]),
            full(*fc3_w.shape), full(1, fc3_b.shape[0]),
        ],
        out_specs=[full(B, ncls), full(1, 9)],
        out_shape=[
            jax.ShapeDtypeStruct((B, ncls), _F32),
            jax.ShapeDtypeStruct((1, 9), _F32),
        ],
        scratch_shapes=[pltpu.VMEM((B, F3), _F32),
                        pltpu.VMEM((F1, F1), _F32),
                        pltpu.VMEM((F2, F2), _F32),
                        pltpu.VMEM((F3, F3), _F32)],
        compiler_params=pltpu.CompilerParams(
            dimension_semantics=("arbitrary",)),
    )(x, conv1_w, conv1_b.reshape(1, F1), conv2_w, conv2_b.reshape(1, F2),
      conv3_w, conv3_b.reshape(1, F3), fc1_w, fc1_b.reshape(1, -1),
      fc2_w, fc2_b.reshape(1, -1), fc3_w, fc3_b.reshape(1, -1))

    return logits, regs.reshape(9)
